# baseline (speedup 1.0000x reference)
import functools

import jax
import jax.numpy as jnp
from jax import lax
from jax.experimental import pallas as pl
from jax.experimental.pallas import tpu as pltpu
from jax.experimental.pallas import tpu_sc as plsc

NC = 2
NS = 16
LANES = 16
CH = 80
NBUF = 4
EBUF = 8


@functools.cache
def _make_sc_agg(N: int, H: int, n_chunks: int):
    mesh = plsc.VectorSubcoreMesh(core_axis_name="c", subcore_axis_name="s",
                                  num_cores=NC, num_subcores=NS)
    rows_per_tile = N // NS
    groups = H // LANES
    zchunk = rows_per_tile // 5

    @functools.partial(
        pl.kernel,
        out_type=jax.ShapeDtypeStruct((NC, NS, rows_per_tile, H), jnp.float32),
        mesh=mesh,
        scratch_types=[
            pltpu.VMEM((EBUF, CH), jnp.int32),
            pltpu.VMEM((EBUF, CH), jnp.int32),
            pltpu.VMEM((EBUF, CH), jnp.float32),
            pltpu.VMEM((NBUF, CH, H), jnp.float32),
            pltpu.VMEM((zchunk, H), jnp.float32),
            pltpu.VMEM_SHARED((N, H), jnp.float32),
            pltpu.VMEM_SHARED((N, H), jnp.float32),
            pltpu.SemaphoreType.DMA((EBUF,)),
            pltpu.SemaphoreType.DMA((NBUF,)),
            pltpu.SemaphoreType.DMA((NBUF,)),
        ],
        compiler_params=pltpu.CompilerParams(use_tc_tiling_on_sc=False),
    )
    def sc_agg(sup_hbm, src_hbm, dst_hbm, w_hbm, out_hbm,
               src_v, dst_v, w_v, rows_v, zbuf_v, sup_sh, acc_sh,
               esem, gsem, ssem):
        cid = lax.axis_index("c")
        sid = lax.axis_index("s")
        wid = sid * NC + cid
        base = sid * rows_per_tile

        def ed_copies(k):
            eb = lax.rem(k, EBUF)
            row = wid * n_chunks + k
            return [pltpu.make_async_copy(src_hbm.at[row], src_v.at[eb],
                                          esem.at[eb]),
                    pltpu.make_async_copy(dst_hbm.at[row], dst_v.at[eb],
                                          esem.at[eb]),
                    pltpu.make_async_copy(w_hbm.at[row], w_v.at[eb],
                                          esem.at[eb])]

        def ed_start(k):
            for cp in ed_copies(k):
                cp.start()

        def ed_wait(k):
            for cp in ed_copies(k):
                cp.wait()

        for k in range(min(EBUF, n_chunks)):
            ed_start(k)
        pltpu.sync_copy(sup_hbm.at[pl.ds(base, rows_per_tile)],
                        sup_sh.at[pl.ds(base, rows_per_tile)])

        def zero_row(i, carry):
            for j in range(groups):
                zbuf_v[i, pl.ds(j * LANES, LANES)] = jnp.zeros((LANES,), jnp.float32)
            return carry
        lax.fori_loop(0, zchunk, zero_row, 0)
        for k in range(5):
            pltpu.sync_copy(zbuf_v, acc_sh.at[pl.ds(base + k * zchunk, zchunk)])
        plsc.subcore_barrier()

        def gather(g):
            return pltpu.make_async_copy(sup_sh.at[src_v.at[lax.rem(g, EBUF)]],
                                         rows_v.at[lax.rem(g, NBUF)],
                                         gsem.at[lax.rem(g, NBUF)])

        def scatter(s):
            return pltpu.make_async_copy(rows_v.at[lax.rem(s, NBUF)],
                                         acc_sh.at[dst_v.at[lax.rem(s, EBUF)]],
                                         ssem.at[lax.rem(s, NBUF)])

        for g in range(min(2, n_chunks)):
            ed_wait(g)
            gather(g).start()

        def chunk_body(c, carry):
            eb = lax.rem(c, EBUF)
            b = lax.rem(c, NBUF)
            gather(c).wait()

            @plsc.parallel_loop(0, CH // LANES, 1, unroll=2)
            def scale_grp(g):
                wv = w_v[eb, pl.ds(g * LANES, LANES)]
                for l in range(LANES):
                    w = wv[l]
                    e = g * LANES + l
                    for j in range(groups):
                        sl = pl.ds(j * LANES, LANES)
                        rows_v[b, e, sl] = rows_v[b, e, sl] * w

            scatter(c).start(add=True)

            @pl.when(c + 2 < n_chunks)
            def _prefetch():
                @pl.when(c >= 2)
                def _drain():
                    scatter(c - 2).wait()

                    @pl.when(c - 2 + EBUF < n_chunks)
                    def _refill():
                        ed_start(c - 2 + EBUF)

                ed_wait(c + 2)
                gather(c + 2).start()
            return carry
        lax.fori_loop(0, n_chunks, chunk_body, 0)

        for j in range(max(0, n_chunks - NBUF), n_chunks):
            scatter(j).wait()

        plsc.subcore_barrier()

        pltpu.sync_copy(acc_sh.at[pl.ds(base, rows_per_tile)], out_hbm.at[cid, sid])

    return sc_agg


def _elu(h):
    return jnp.where(h > 0, h, jnp.exp(jnp.minimum(h, 0.0)) - 1.0)


def _mm_kernel(x_ref, w_ref, o_ref):
    o_ref[...] = jax.lax.dot_general(
        x_ref[...], w_ref[...], (((1,), (0,)), ((), ())),
        preferred_element_type=jnp.float32)


def _tc_matmul(x, W, bm):
    M, D = x.shape
    H = W.shape[1]
    return pl.pallas_call(
        _mm_kernel,
        grid=(M // bm,),
        in_specs=[pl.BlockSpec((bm, D), lambda i: (i, 0)),
                  pl.BlockSpec((D, H), lambda i: (0, 0))],
        out_specs=pl.BlockSpec((bm, H), lambda i: (i, 0)),
        out_shape=jax.ShapeDtypeStruct((M, H), jnp.float32),
    )(x, W)


def _combine_mm_kernel(p_ref, w_ref, o_ref):
    h = _elu(p_ref[0] + p_ref[1])
    o_ref[...] = jax.lax.dot_general(
        h, w_ref[...], (((1,), (0,)), ((), ())),
        preferred_element_type=jnp.float32)


def _tc_combine_matmul(p, W, bm):
    _, M, D = p.shape
    H = W.shape[1]
    return pl.pallas_call(
        _combine_mm_kernel,
        grid=(M // bm,),
        in_specs=[pl.BlockSpec((2, bm, D), lambda i: (0, i, 0)),
                  pl.BlockSpec((D, H), lambda i: (0, 0))],
        out_specs=pl.BlockSpec((bm, H), lambda i: (i, 0)),
        out_shape=jax.ShapeDtypeStruct((M, H), jnp.float32),
    )(p, W)


def _finish_kernel(q_ref, o_ref):
    mu = _elu(q_ref[0] + q_ref[1])
    norm = jnp.sqrt(jnp.sum(mu * mu, axis=1, keepdims=True))
    o_ref[...] = mu / jnp.maximum(norm, 1e-12)


def _tc_finish(q, bm):
    _, M, H = q.shape
    return pl.pallas_call(
        _finish_kernel,
        grid=(M // bm,),
        in_specs=[pl.BlockSpec((2, bm, H), lambda i: (0, i, 0))],
        out_specs=pl.BlockSpec((bm, H), lambda i: (i, 0)),
        out_shape=jax.ShapeDtypeStruct((M, H), jnp.float32),
    )(q)


@jax.jit
def kernel(x, edge_index, edge_weight, W1, W2):
    N, D = x.shape
    E = edge_index.shape[1]
    H1 = W1.shape[1]
    H2 = W2.shape[1]

    NW = NC * NS
    n_chunks = pl.cdiv(E, NW * CH)
    Ep = NW * n_chunks * CH
    pad = Ep - E

    src = edge_index[0].astype(jnp.int32)
    dst = edge_index[1].astype(jnp.int32)
    w = edge_weight.astype(jnp.float32)
    if pad:
        zi = jnp.zeros((pad,), jnp.int32)
        src = jnp.concatenate([src, zi])
        dst = jnp.concatenate([dst, zi])
        w = jnp.concatenate([w, jnp.zeros((pad,), jnp.float32)])
    src = src.reshape(NW * n_chunks, CH)
    dst = dst.reshape(NW * n_chunks, CH)
    w = w.reshape(NW * n_chunks, CH)

    sc_agg1 = _make_sc_agg(N, H1, n_chunks)
    sc_agg2 = _make_sc_agg(N, H2, n_chunks)

    support1 = _tc_matmul(x, W1, bm=1000)
    p1 = sc_agg1(support1, src, dst, w).reshape(NC, N, H1)
    support2 = _tc_combine_matmul(p1, W2, bm=1000)
    p2 = sc_agg2(support2, src, dst, w).reshape(NC, N, H2)
    z = _tc_finish(p2, bm=1000)
    return z


# speedup vs baseline: 1.1126x; 1.1126x over previous
import functools

import jax
import jax.numpy as jnp
from jax import lax
from jax.experimental import pallas as pl
from jax.experimental.pallas import tpu as pltpu
from jax.experimental.pallas import tpu_sc as plsc

NC = 2
NS = 16
LANES = 16
CH = 128
NBUF = 4
EBUF = 8


@functools.cache
def _make_sc_agg(N: int, H: int, n_chunks: int):
    mesh = plsc.VectorSubcoreMesh(core_axis_name="c", subcore_axis_name="s",
                                  num_cores=NC, num_subcores=NS)
    rows_per_tile = N // NS
    groups = H // LANES
    zchunk = rows_per_tile // 5

    @functools.partial(
        pl.kernel,
        out_type=jax.ShapeDtypeStruct((NC, NS, rows_per_tile, H), jnp.float32),
        mesh=mesh,
        scratch_types=[
            pltpu.VMEM((EBUF, CH), jnp.int32),
            pltpu.VMEM((EBUF, CH), jnp.int32),
            pltpu.VMEM((EBUF, CH), jnp.float32),
            pltpu.VMEM((NBUF, CH, H), jnp.float32),
            pltpu.VMEM((zchunk, H), jnp.float32),
            pltpu.VMEM_SHARED((N, H), jnp.float32),
            pltpu.VMEM_SHARED((N, H), jnp.float32),
            pltpu.SemaphoreType.DMA((EBUF,)),
            pltpu.SemaphoreType.DMA((NBUF,)),
            pltpu.SemaphoreType.DMA((NBUF,)),
        ],
        compiler_params=pltpu.CompilerParams(use_tc_tiling_on_sc=False),
    )
    def sc_agg(sup_hbm, src_hbm, dst_hbm, w_hbm, out_hbm,
               src_v, dst_v, w_v, rows_v, zbuf_v, sup_sh, acc_sh,
               esem, gsem, ssem):
        cid = lax.axis_index("c")
        sid = lax.axis_index("s")
        wid = sid * NC + cid
        base = sid * rows_per_tile

        def ed_copies(k):
            eb = lax.rem(k, EBUF)
            row = wid * n_chunks + k
            return [pltpu.make_async_copy(src_hbm.at[row], src_v.at[eb],
                                          esem.at[eb]),
                    pltpu.make_async_copy(dst_hbm.at[row], dst_v.at[eb],
                                          esem.at[eb]),
                    pltpu.make_async_copy(w_hbm.at[row], w_v.at[eb],
                                          esem.at[eb])]

        def ed_start(k):
            for cp in ed_copies(k):
                cp.start()

        def ed_wait(k):
            for cp in ed_copies(k):
                cp.wait()

        for k in range(min(EBUF, n_chunks)):
            ed_start(k)
        pltpu.sync_copy(sup_hbm.at[pl.ds(base, rows_per_tile)],
                        sup_sh.at[pl.ds(base, rows_per_tile)])

        def zero_row(i, carry):
            for j in range(groups):
                zbuf_v[i, pl.ds(j * LANES, LANES)] = jnp.zeros((LANES,), jnp.float32)
            return carry
        lax.fori_loop(0, zchunk, zero_row, 0)
        for k in range(5):
            pltpu.sync_copy(zbuf_v, acc_sh.at[pl.ds(base + k * zchunk, zchunk)])
        plsc.subcore_barrier()

        def gather(g):
            return pltpu.make_async_copy(sup_sh.at[src_v.at[lax.rem(g, EBUF)]],
                                         rows_v.at[lax.rem(g, NBUF)],
                                         gsem.at[lax.rem(g, NBUF)])

        def scatter(s):
            return pltpu.make_async_copy(rows_v.at[lax.rem(s, NBUF)],
                                         acc_sh.at[dst_v.at[lax.rem(s, EBUF)]],
                                         ssem.at[lax.rem(s, NBUF)])

        for g in range(min(2, n_chunks)):
            ed_wait(g)
            gather(g).start()

        def chunk_body(c, carry):
            eb = lax.rem(c, EBUF)
            b = lax.rem(c, NBUF)
            gather(c).wait()

            @plsc.parallel_loop(0, CH // LANES, 1, unroll=2)
            def scale_grp(g):
                wv = w_v[eb, pl.ds(g * LANES, LANES)]
                for l in range(LANES):
                    w = wv[l]
                    e = g * LANES + l
                    for j in range(groups):
                        sl = pl.ds(j * LANES, LANES)
                        rows_v[b, e, sl] = rows_v[b, e, sl] * w

            scatter(c).start(add=True)

            @pl.when(c + 2 < n_chunks)
            def _prefetch():
                @pl.when(c >= 2)
                def _drain():
                    scatter(c - 2).wait()

                    @pl.when(c - 2 + EBUF < n_chunks)
                    def _refill():
                        ed_start(c - 2 + EBUF)

                ed_wait(c + 2)
                gather(c + 2).start()
            return carry
        lax.fori_loop(0, n_chunks, chunk_body, 0)

        for j in range(max(0, n_chunks - NBUF), n_chunks):
            scatter(j).wait()

        plsc.subcore_barrier()

        pltpu.sync_copy(acc_sh.at[pl.ds(base, rows_per_tile)], out_hbm.at[cid, sid])

    return sc_agg


def _elu(h):
    return jnp.where(h > 0, h, jnp.exp(jnp.minimum(h, 0.0)) - 1.0)


def _mm_kernel(x_ref, w_ref, o_ref):
    o_ref[...] = jax.lax.dot_general(
        x_ref[...], w_ref[...], (((1,), (0,)), ((), ())),
        preferred_element_type=jnp.float32)


def _tc_matmul(x, W, bm):
    M, D = x.shape
    H = W.shape[1]
    return pl.pallas_call(
        _mm_kernel,
        grid=(M // bm,),
        in_specs=[pl.BlockSpec((bm, D), lambda i: (i, 0)),
                  pl.BlockSpec((D, H), lambda i: (0, 0))],
        out_specs=pl.BlockSpec((bm, H), lambda i: (i, 0)),
        out_shape=jax.ShapeDtypeStruct((M, H), jnp.float32),
    )(x, W)


def _combine_mm_kernel(p_ref, w_ref, o_ref):
    h = _elu(p_ref[0] + p_ref[1])
    o_ref[...] = jax.lax.dot_general(
        h, w_ref[...], (((1,), (0,)), ((), ())),
        preferred_element_type=jnp.float32)


def _tc_combine_matmul(p, W, bm):
    _, M, D = p.shape
    H = W.shape[1]
    return pl.pallas_call(
        _combine_mm_kernel,
        grid=(M // bm,),
        in_specs=[pl.BlockSpec((2, bm, D), lambda i: (0, i, 0)),
                  pl.BlockSpec((D, H), lambda i: (0, 0))],
        out_specs=pl.BlockSpec((bm, H), lambda i: (i, 0)),
        out_shape=jax.ShapeDtypeStruct((M, H), jnp.float32),
    )(p, W)


def _finish_kernel(q_ref, o_ref):
    mu = _elu(q_ref[0] + q_ref[1])
    norm = jnp.sqrt(jnp.sum(mu * mu, axis=1, keepdims=True))
    o_ref[...] = mu / jnp.maximum(norm, 1e-12)


def _tc_finish(q, bm):
    _, M, H = q.shape
    return pl.pallas_call(
        _finish_kernel,
        grid=(M // bm,),
        in_specs=[pl.BlockSpec((2, bm, H), lambda i: (0, i, 0))],
        out_specs=pl.BlockSpec((bm, H), lambda i: (i, 0)),
        out_shape=jax.ShapeDtypeStruct((M, H), jnp.float32),
    )(q)


@jax.jit
def kernel(x, edge_index, edge_weight, W1, W2):
    N, D = x.shape
    E = edge_index.shape[1]
    H1 = W1.shape[1]
    H2 = W2.shape[1]

    NW = NC * NS
    n_chunks = pl.cdiv(E, NW * CH)
    Ep = NW * n_chunks * CH
    pad = Ep - E

    src = edge_index[0].astype(jnp.int32)
    dst = edge_index[1].astype(jnp.int32)
    w = edge_weight.astype(jnp.float32)
    if pad:
        zi = jnp.zeros((pad,), jnp.int32)
        src = jnp.concatenate([src, zi])
        dst = jnp.concatenate([dst, zi])
        w = jnp.concatenate([w, jnp.zeros((pad,), jnp.float32)])
    src = src.reshape(NW * n_chunks, CH)
    dst = dst.reshape(NW * n_chunks, CH)
    w = w.reshape(NW * n_chunks, CH)

    sc_agg1 = _make_sc_agg(N, H1, n_chunks)
    sc_agg2 = _make_sc_agg(N, H2, n_chunks)

    support1 = _tc_matmul(x, W1, bm=1000)
    p1 = sc_agg1(support1, src, dst, w).reshape(NC, N, H1)
    support2 = _tc_combine_matmul(p1, W2, bm=1000)
    p2 = sc_agg2(support2, src, dst, w).reshape(NC, N, H2)
    z = _tc_finish(p2, bm=1000)
    return z
